# baseline (speedup 1.0000x reference)
import functools

import jax
import jax.numpy as jnp
from jax import lax
from jax.experimental import pallas as pl
from jax.experimental.pallas import tpu as pltpu
from jax.experimental.pallas import tpu_sc as plsc

N = 10000
E = 320000
NFEAT = 128
NHID = 128
NCLASS = 64

K = 125
NROWS = E // K
CPW = NROWS // 32
DW = 16
NWIND = CPW // DW
NPAD = 10112
STRIPE = NPAD // 16

_MESH = plsc.VectorSubcoreMesh(core_axis_name="c", subcore_axis_name="s")


def _deg_partials(dst2d, ones_blk, zeros_blk):
  @functools.partial(
      pl.kernel,
      out_type=jax.ShapeDtypeStruct((2, NPAD, 128), jnp.float32),
      mesh=_MESH,
      scratch_types=[
          pltpu.VMEM((CPW, K), jnp.int32),
          pltpu.VMEM((K, 128), jnp.float32),
          pltpu.VMEM_SHARED((NPAD, 128), jnp.float32),
      ],
  )
  def deg_kernel(dst_hbm, ones_hbm, zeros_hbm, out_hbm, dst_v, ones_v, acc_sh):
    cid = lax.axis_index("c")
    sid = lax.axis_index("s")
    row0 = (cid * 16 + sid) * CPW
    pltpu.sync_copy(dst_hbm.at[pl.ds(row0, CPW)], dst_v)
    pltpu.sync_copy(ones_hbm, ones_v)
    stripe = sid * STRIPE
    pltpu.sync_copy(zeros_hbm, acc_sh.at[pl.ds(stripe, STRIPE)])
    plsc.subcore_barrier()

    @pl.loop(0, CPW)
    def _(j):
      pltpu.sync_copy(ones_v, acc_sh.at[dst_v.at[j]], add=True)

    plsc.subcore_barrier()
    pltpu.sync_copy(acc_sh.at[pl.ds(stripe, STRIPE)],
                    out_hbm.at[cid, pl.ds(stripe, STRIPE)])

  return deg_kernel(dst2d, ones_blk, zeros_blk)


def _make_agg(D):
  @functools.partial(
      pl.kernel,
      out_type=jax.ShapeDtypeStruct((2, NPAD, D), jnp.float32),
      mesh=_MESH,
      scratch_types=[
          pltpu.VMEM((CPW, K), jnp.int32),
          pltpu.VMEM((DW, K), jnp.int32),
          pltpu.VMEM((K, D), jnp.float32),
          pltpu.VMEM((K, D), jnp.float32),
          pltpu.VMEM_SHARED((NPAD, D), jnp.float32),
          pltpu.SemaphoreType.DMA,
          pltpu.SemaphoreType.DMA,
          pltpu.SemaphoreType.DMA,
          pltpu.SemaphoreType.DMA,
      ],
  )
  def agg_kernel(hs_hbm, src_hbm, dst_hbm, zeros_hbm, out_hbm,
                 src_v, dst_v, buf0, buf1, acc_sh, sem0, sem1, ssem0, ssem1):
    cid = lax.axis_index("c")
    sid = lax.axis_index("s")
    row0 = (cid * 16 + sid) * CPW
    pltpu.sync_copy(src_hbm.at[pl.ds(row0, CPW)], src_v)
    stripe = sid * STRIPE
    pltpu.sync_copy(zeros_hbm, acc_sh.at[pl.ds(stripe, STRIPE)])
    plsc.subcore_barrier()

    pltpu.async_copy(hs_hbm.at[src_v.at[0]], buf0, sem0)
    pltpu.async_copy(hs_hbm.at[src_v.at[1]], buf1, sem1)

    @pl.loop(0, NWIND)
    def _(w):
      pltpu.sync_copy(dst_hbm.at[pl.ds(row0 + w * DW, DW)], dst_v)

      @pl.loop(0, DW // 2)
      def _(p):
        j = w * DW + 2 * p
        pltpu.make_async_copy(hs_hbm.at[src_v.at[j]], buf0, sem0).wait()
        pltpu.async_copy(buf0, acc_sh.at[dst_v.at[2 * p]], ssem0, add=True)

        pltpu.make_async_copy(hs_hbm.at[src_v.at[j + 1]], buf1, sem1).wait()
        pltpu.async_copy(buf1, acc_sh.at[dst_v.at[2 * p + 1]], ssem1, add=True)

        pltpu.make_async_copy(buf0, acc_sh.at[dst_v.at[2 * p]], ssem0).wait()

        @pl.when(j + 2 < CPW)
        def _():
          pltpu.async_copy(hs_hbm.at[src_v.at[j + 2]], buf0, sem0)

        pltpu.make_async_copy(buf1, acc_sh.at[dst_v.at[2 * p + 1]], ssem1).wait()

        @pl.when(j + 3 < CPW)
        def _():
          pltpu.async_copy(hs_hbm.at[src_v.at[j + 3]], buf1, sem1)

    plsc.subcore_barrier()
    pltpu.sync_copy(acc_sh.at[pl.ds(stripe, STRIPE)],
                    out_hbm.at[cid, pl.ds(stripe, STRIPE)])

  return agg_kernel


_agg = _make_agg(NHID)


_R = 2000


def _tc_layer1(x, W1, degp):
  def body(x_ref, w_ref, deg_ref, hs_ref, dinv_ref):
    h = jnp.dot(x_ref[...], w_ref[...], preferred_element_type=jnp.float32)
    deg = deg_ref[0, :, 0:1] + deg_ref[1, :, 0:1] + 1.0
    dinv = lax.rsqrt(deg)
    hs_ref[...] = h * dinv
    dinv_ref[...] = jnp.broadcast_to(dinv, dinv_ref.shape)

  return pl.pallas_call(
      body,
      grid=(N // _R,),
      in_specs=[pl.BlockSpec((_R, NFEAT), lambda i: (i, 0)),
                pl.BlockSpec((NFEAT, NHID), lambda i: (0, 0)),
                pl.BlockSpec((2, _R, 128), lambda i: (0, i, 0))],
      out_specs=[pl.BlockSpec((_R, NHID), lambda i: (i, 0)),
                 pl.BlockSpec((_R, NHID), lambda i: (i, 0))],
      out_shape=[jax.ShapeDtypeStruct((N, NHID), jnp.float32),
                 jax.ShapeDtypeStruct((N, NHID), jnp.float32)],
  )(x, W1, degp)


def _tc_layer2(acc1, hs1, dinvb, b1, W2):
  def body(acc_ref, hs_ref, dinv_ref, b_ref, w_ref, o_ref):
    s = acc_ref[0] + acc_ref[1] + hs_ref[...]
    z = jnp.maximum(dinv_ref[...] * s + b_ref[...], 0.0)
    h2 = jnp.dot(z, w_ref[...], preferred_element_type=jnp.float32)
    o_ref[...] = jnp.concatenate(
        [h2 * dinv_ref[:, :NCLASS], jnp.zeros_like(h2)], axis=1)

  return pl.pallas_call(
      body,
      grid=(N // _R,),
      in_specs=[pl.BlockSpec((2, _R, NHID), lambda i: (0, i, 0)),
                pl.BlockSpec((_R, NHID), lambda i: (i, 0)),
                pl.BlockSpec((_R, NHID), lambda i: (i, 0)),
                pl.BlockSpec((1, NHID), lambda i: (0, 0)),
                pl.BlockSpec((NHID, NCLASS), lambda i: (0, 0))],
      out_specs=pl.BlockSpec((_R, NHID), lambda i: (i, 0)),
      out_shape=jax.ShapeDtypeStruct((N, NHID), jnp.float32),
  )(acc1, hs1, dinvb, b1, W2)


def _tc_out(acc2, hs2, dinvb, b2):
  def body(acc_ref, hs_ref, dinv_ref, b_ref, o_ref):
    t = dinv_ref[:, :NCLASS] * (
        acc_ref[0, :, :NCLASS] + acc_ref[1, :, :NCLASS] + hs_ref[:, :NCLASS])
    t = t + b_ref[...]
    m = jnp.max(t, axis=1, keepdims=True)
    e = t - m
    lse = jnp.log(jnp.sum(jnp.exp(e), axis=1, keepdims=True))
    o_ref[...] = e - lse

  return pl.pallas_call(
      body,
      grid=(N // _R,),
      in_specs=[pl.BlockSpec((2, _R, NHID), lambda i: (0, i, 0)),
                pl.BlockSpec((_R, NHID), lambda i: (i, 0)),
                pl.BlockSpec((_R, NHID), lambda i: (i, 0)),
                pl.BlockSpec((1, NCLASS), lambda i: (0, 0))],
      out_specs=pl.BlockSpec((_R, NCLASS), lambda i: (i, 0)),
      out_shape=jax.ShapeDtypeStruct((N, NCLASS), jnp.float32),
  )(acc2, hs2, dinvb, b2)


def kernel(x, edge_index, W1, b1, W2, b2):
  src2d = edge_index[0].reshape(NROWS, K)
  dst2d = edge_index[1].reshape(NROWS, K)
  ones_blk = jnp.ones((K, 128), jnp.float32)
  zeros16 = jnp.zeros((STRIPE, 128), jnp.float32)
  zeros128 = jnp.zeros((STRIPE, NHID), jnp.float32)

  degp = _deg_partials(dst2d, ones_blk, zeros16)
  hs1, dinvb = _tc_layer1(x, W1, degp)
  acc1 = _agg(hs1, src2d, dst2d, zeros128)
  hs2 = _tc_layer2(acc1, hs1, dinvb, b1.reshape(1, NHID), W2)
  acc2 = _agg(hs2, src2d, dst2d, zeros128)
  return _tc_out(acc2, hs2, dinvb, b2.reshape(1, NCLASS))


# speedup vs baseline: 1.1939x; 1.1939x over previous
import functools

import jax
import jax.numpy as jnp
from jax import lax
from jax.experimental import pallas as pl
from jax.experimental.pallas import tpu as pltpu
from jax.experimental.pallas import tpu_sc as plsc

N = 10000
E = 320000
NFEAT = 128
NHID = 128
NCLASS = 64

K = 125
NROWS = E // K
CPW = NROWS // 32
DW = 16
NWIND = CPW // DW
NPAD = 10112
STRIPE = NPAD // 16

_MESH = plsc.VectorSubcoreMesh(core_axis_name="c", subcore_axis_name="s")


def _deg_partials(dst2d, ones_blk, zeros_blk):
  @functools.partial(
      pl.kernel,
      out_type=jax.ShapeDtypeStruct((2, NPAD, 128), jnp.float32),
      mesh=_MESH,
      scratch_types=[
          pltpu.VMEM((CPW, K), jnp.int32),
          pltpu.VMEM((K, 128), jnp.float32),
          pltpu.VMEM_SHARED((NPAD, 128), jnp.float32),
      ],
  )
  def deg_kernel(dst_hbm, ones_hbm, zeros_hbm, out_hbm, dst_v, ones_v, acc_sh):
    cid = lax.axis_index("c")
    sid = lax.axis_index("s")
    row0 = (cid * 16 + sid) * CPW
    pltpu.sync_copy(dst_hbm.at[pl.ds(row0, CPW)], dst_v)
    pltpu.sync_copy(ones_hbm, ones_v)
    stripe = sid * STRIPE
    pltpu.sync_copy(zeros_hbm, acc_sh.at[pl.ds(stripe, STRIPE)])
    plsc.subcore_barrier()

    @pl.loop(0, CPW)
    def _(j):
      pltpu.sync_copy(ones_v, acc_sh.at[dst_v.at[j]], add=True)

    plsc.subcore_barrier()
    pltpu.sync_copy(acc_sh.at[pl.ds(stripe, STRIPE)],
                    out_hbm.at[cid, pl.ds(stripe, STRIPE)])

  return deg_kernel(dst2d, ones_blk, zeros_blk)


def _make_agg(D):
  @functools.partial(
      pl.kernel,
      out_type=jax.ShapeDtypeStruct((2, NPAD, D), jnp.float32),
      mesh=_MESH,
      scratch_types=[
          pltpu.VMEM((CPW, K), jnp.int32),
          pltpu.VMEM((DW, K), jnp.int32),
          pltpu.VMEM((K, D), jnp.float32),
          pltpu.VMEM((K, D), jnp.float32),
          pltpu.VMEM_SHARED((NPAD, D), jnp.float32),
          pltpu.SemaphoreType.DMA,
          pltpu.SemaphoreType.DMA,
      ],
  )
  def agg_kernel(hs_hbm, src_hbm, dst_hbm, zeros_hbm, out_hbm,
                 src_v, dst_v, buf0, buf1, acc_sh, sem0, sem1):
    cid = lax.axis_index("c")
    sid = lax.axis_index("s")
    row0 = (cid * 16 + sid) * CPW
    pltpu.sync_copy(src_hbm.at[pl.ds(row0, CPW)], src_v)
    stripe = sid * STRIPE
    pltpu.sync_copy(zeros_hbm, acc_sh.at[pl.ds(stripe, STRIPE)])
    plsc.subcore_barrier()

    pltpu.async_copy(hs_hbm.at[src_v.at[0]], buf0, sem0)
    pltpu.async_copy(hs_hbm.at[src_v.at[1]], buf1, sem1)

    @pl.loop(0, NWIND)
    def _(w):
      pltpu.sync_copy(dst_hbm.at[pl.ds(row0 + w * DW, DW)], dst_v)

      @pl.loop(0, DW // 2)
      def _(p):
        j = w * DW + 2 * p
        pltpu.make_async_copy(hs_hbm.at[src_v.at[j]], buf0, sem0).wait()
        pltpu.sync_copy(buf0, acc_sh.at[dst_v.at[2 * p]], add=True)

        @pl.when(j + 2 < CPW)
        def _():
          pltpu.async_copy(hs_hbm.at[src_v.at[j + 2]], buf0, sem0)

        pltpu.make_async_copy(hs_hbm.at[src_v.at[j + 1]], buf1, sem1).wait()
        pltpu.sync_copy(buf1, acc_sh.at[dst_v.at[2 * p + 1]], add=True)

        @pl.when(j + 3 < CPW)
        def _():
          pltpu.async_copy(hs_hbm.at[src_v.at[j + 3]], buf1, sem1)

    plsc.subcore_barrier()
    pltpu.sync_copy(acc_sh.at[pl.ds(stripe, STRIPE)],
                    out_hbm.at[cid, pl.ds(stripe, STRIPE)])

  return agg_kernel


_agg = _make_agg(NHID)


_R = 2000


def _tc_layer1(x, W1, degp):
  def body(x_ref, w_ref, deg_ref, hs_ref, dinv_ref):
    h = jnp.dot(x_ref[...], w_ref[...], preferred_element_type=jnp.float32)
    deg = deg_ref[0, :, 0:1] + deg_ref[1, :, 0:1] + 1.0
    dinv = lax.rsqrt(deg)
    hs_ref[...] = h * dinv
    dinv_ref[...] = jnp.broadcast_to(dinv, dinv_ref.shape)

  return pl.pallas_call(
      body,
      grid=(N // _R,),
      in_specs=[pl.BlockSpec((_R, NFEAT), lambda i: (i, 0)),
                pl.BlockSpec((NFEAT, NHID), lambda i: (0, 0)),
                pl.BlockSpec((2, _R, 128), lambda i: (0, i, 0))],
      out_specs=[pl.BlockSpec((_R, NHID), lambda i: (i, 0)),
                 pl.BlockSpec((_R, NHID), lambda i: (i, 0))],
      out_shape=[jax.ShapeDtypeStruct((N, NHID), jnp.float32),
                 jax.ShapeDtypeStruct((N, NHID), jnp.float32)],
  )(x, W1, degp)


def _tc_layer2(acc1, hs1, dinvb, b1, W2):
  def body(acc_ref, hs_ref, dinv_ref, b_ref, w_ref, o_ref):
    s = acc_ref[0] + acc_ref[1] + hs_ref[...]
    z = jnp.maximum(dinv_ref[...] * s + b_ref[...], 0.0)
    h2 = jnp.dot(z, w_ref[...], preferred_element_type=jnp.float32)
    o_ref[...] = jnp.concatenate(
        [h2 * dinv_ref[:, :NCLASS], jnp.zeros_like(h2)], axis=1)

  return pl.pallas_call(
      body,
      grid=(N // _R,),
      in_specs=[pl.BlockSpec((2, _R, NHID), lambda i: (0, i, 0)),
                pl.BlockSpec((_R, NHID), lambda i: (i, 0)),
                pl.BlockSpec((_R, NHID), lambda i: (i, 0)),
                pl.BlockSpec((1, NHID), lambda i: (0, 0)),
                pl.BlockSpec((NHID, NCLASS), lambda i: (0, 0))],
      out_specs=pl.BlockSpec((_R, NHID), lambda i: (i, 0)),
      out_shape=jax.ShapeDtypeStruct((N, NHID), jnp.float32),
  )(acc1, hs1, dinvb, b1, W2)


def _tc_out(acc2, hs2, dinvb, b2):
  def body(acc_ref, hs_ref, dinv_ref, b_ref, o_ref):
    t = dinv_ref[:, :NCLASS] * (
        acc_ref[0, :, :NCLASS] + acc_ref[1, :, :NCLASS] + hs_ref[:, :NCLASS])
    t = t + b_ref[...]
    m = jnp.max(t, axis=1, keepdims=True)
    e = t - m
    lse = jnp.log(jnp.sum(jnp.exp(e), axis=1, keepdims=True))
    o_ref[...] = e - lse

  return pl.pallas_call(
      body,
      grid=(N // _R,),
      in_specs=[pl.BlockSpec((2, _R, NHID), lambda i: (0, i, 0)),
                pl.BlockSpec((_R, NHID), lambda i: (i, 0)),
                pl.BlockSpec((_R, NHID), lambda i: (i, 0)),
                pl.BlockSpec((1, NCLASS), lambda i: (0, 0))],
      out_specs=pl.BlockSpec((_R, NCLASS), lambda i: (i, 0)),
      out_shape=jax.ShapeDtypeStruct((N, NCLASS), jnp.float32),
  )(acc2, hs2, dinvb, b2)


def kernel(x, edge_index, W1, b1, W2, b2):
  src2d = edge_index[0].reshape(NROWS, K)
  dst2d = edge_index[1].reshape(NROWS, K)
  ones_blk = jnp.ones((K, 128), jnp.float32)
  zeros16 = jnp.zeros((STRIPE, 128), jnp.float32)
  zeros128 = jnp.zeros((STRIPE, NHID), jnp.float32)

  degp = _deg_partials(dst2d, ones_blk, zeros16)
  hs1, dinvb = _tc_layer1(x, W1, degp)
  acc1 = _agg(hs1, src2d, dst2d, zeros128)
  hs2 = _tc_layer2(acc1, hs1, dinvb, b1.reshape(1, NHID), W2)
  acc2 = _agg(hs2, src2d, dst2d, zeros128)
  return _tc_out(acc2, hs2, dinvb, b2.reshape(1, NCLASS))
